# baseline (speedup 1.0000x reference)
import functools

import jax
import jax.numpy as jnp
from jax import lax
from jax.experimental import pallas as pl
from jax.experimental.pallas import tpu as pltpu
from jax.experimental.pallas import tpu_sc as plsc

EMB = 1024
BATCH = 4
SEQ = 4096
NUM_SEG = 3
NUM_ROWS = BATCH * SEQ

SC_ROWS = 8192
TC_ROWS = NUM_ROWS - SC_ROWS

NC = 2
NS = 16
NW = NC * NS
RPW = SC_ROWS // NW
GRP = 16
NG = RPW // GRP
LAG = 8

BR = 512
NBLK = TC_ROWS // BR

_mesh = plsc.VectorSubcoreMesh(core_axis_name="c", subcore_axis_name="s")


@functools.partial(
    pl.kernel,
    mesh=_mesh,
    out_type=jax.ShapeDtypeStruct((SC_ROWS, EMB), jnp.float32),
    scratch_types=[
        pltpu.VMEM((RPW,), jnp.int32),
        pltpu.VMEM((NUM_SEG, EMB), jnp.float32),
        pltpu.SemaphoreType.DMA,
    ],
)
def _seg_gather_sc(seg_hbm, table_hbm, out_hbm, idx_v, table_v, sem):
    wid = lax.axis_index("s") * NC + lax.axis_index("c")
    base = wid * RPW

    pltpu.sync_copy(seg_hbm.at[pl.ds(base, RPW)], idx_v)
    pltpu.sync_copy(table_hbm, table_v)

    def issue_group(g):
        v = idx_v[pl.ds(g * GRP, GRP)]
        for j in range(GRP):
            pltpu.async_copy(table_v.at[v[j]], out_hbm.at[base + g * GRP + j],
                             sem)

    def wait_group(_g, _):
        pltpu.make_async_copy(out_hbm.at[pl.ds(base, GRP)],
                              out_hbm.at[pl.ds(base, GRP)], sem).wait()
        return 0

    def step(g, _):
        issue_group(g)
        return lax.cond(g >= LAG, lambda: wait_group(g, 0), lambda: 0)

    lax.fori_loop(0, NG, step, 0, unroll=False)
    lax.fori_loop(0, LAG, wait_group, 0, unroll=False)


def _tc_body(seg_ref, table_ref, out_ref):
    sval = seg_ref[...]
    cls = lax.broadcasted_iota(jnp.int32, (BR, NUM_SEG), 1)
    onehot = (sval == cls).astype(jnp.float32)
    out_ref[...] = jnp.dot(onehot, table_ref[...],
                           preferred_element_type=jnp.float32)


def _tc_lookup(seg_col, table):
    return pl.pallas_call(
        _tc_body,
        grid=(NBLK,),
        in_specs=[
            pl.BlockSpec((BR, 1), lambda i: (i, 0)),
            pl.BlockSpec((NUM_SEG, EMB), lambda i: (0, 0)),
        ],
        out_specs=pl.BlockSpec((BR, EMB), lambda i: (i, 0)),
        out_shape=jax.ShapeDtypeStruct((TC_ROWS, EMB), jnp.float32),
    )(seg_col, table)


def kernel(unused, seg, table):
    del unused
    seg_flat = seg.reshape(NUM_ROWS)
    out_tc = _tc_lookup(seg_flat[SC_ROWS:].reshape(TC_ROWS, 1), table)
    out_sc = _seg_gather_sc(seg_flat[:SC_ROWS], table)
    out = jnp.concatenate([out_sc[:SC_ROWS - 1], out_tc[:1] * 0.0
                           + out_sc[SC_ROWS - 1:]], axis=0)
    out = jnp.concatenate([out, out_tc[:0]], axis=0)
    full = jnp.broadcast_to(out, (SC_ROWS, EMB))
    return jnp.concatenate([full, full], axis=0).reshape(BATCH, SEQ, EMB)


# speedup vs baseline: 4.7465x; 4.7465x over previous
import functools

import jax
import jax.numpy as jnp
from jax import lax
from jax.experimental import pallas as pl
from jax.experimental.pallas import tpu as pltpu
from jax.experimental.pallas import tpu_sc as plsc

EMB = 1024
BATCH = 4
SEQ = 4096
NUM_SEG = 3
NUM_ROWS = BATCH * SEQ
NC = 2
NS = 16
NW = NC * NS
RPW = NUM_ROWS // NW
GRP = 16
NG = RPW // GRP
LAG = 8

_mesh = plsc.VectorSubcoreMesh(core_axis_name="c", subcore_axis_name="s")


@functools.partial(
    pl.kernel,
    mesh=_mesh,
    out_type=jax.ShapeDtypeStruct((NUM_ROWS, EMB), jnp.float32),
    scratch_types=[
        pltpu.VMEM((RPW,), jnp.int32),
        pltpu.VMEM((NUM_SEG, EMB), jnp.float32),
        pltpu.SemaphoreType.DMA,
        pltpu.SemaphoreType.DMA,
    ],
)
def _seg_gather(seg_hbm, table_hbm, out_hbm, idx_v, table_v, sem, ssem):
    wid = lax.axis_index("s") * NC + lax.axis_index("c")
    base = wid * RPW

    pltpu.async_copy(seg_hbm.at[pl.ds(base, RPW)], idx_v, ssem)
    pltpu.async_copy(table_hbm, table_v, ssem)
    pltpu.make_async_copy(seg_hbm.at[pl.ds(base, RPW)], idx_v, ssem).wait()
    pltpu.make_async_copy(table_hbm, table_v, ssem).wait()

    def issue_group(g):
        v = idx_v[pl.ds(g * GRP, GRP)]
        for j in range(GRP):
            pltpu.async_copy(table_v.at[v[j]], out_hbm.at[base + g * GRP + j],
                             sem)

    def wait_group(_g, _):
        pltpu.make_async_copy(out_hbm.at[pl.ds(base, GRP)],
                              out_hbm.at[pl.ds(base, GRP)], sem).wait()
        return 0

    def step(g, _):
        issue_group(g)
        return lax.cond(g >= LAG, lambda: wait_group(g, 0), lambda: 0)

    lax.fori_loop(0, NG, step, 0, unroll=False)
    lax.fori_loop(0, LAG, wait_group, 0, unroll=False)


def kernel(unused, seg, table):
    del unused
    out = _seg_gather(seg.reshape(NUM_ROWS), table)
    return out.reshape(BATCH, SEQ, EMB)


# speedup vs baseline: 4.7569x; 1.0022x over previous
import functools

import jax
import jax.numpy as jnp
from jax import lax
from jax.experimental import pallas as pl
from jax.experimental.pallas import tpu as pltpu
from jax.experimental.pallas import tpu_sc as plsc

EMB = 1024
BATCH = 4
SEQ = 4096
NUM_SEG = 3
NUM_ROWS = BATCH * SEQ
NC = 2
NS = 16
NW = NC * NS
RPW = NUM_ROWS // NW
GRP = 16
NG = RPW // GRP
LAG = 16

_mesh = plsc.VectorSubcoreMesh(core_axis_name="c", subcore_axis_name="s")


@functools.partial(
    pl.kernel,
    mesh=_mesh,
    out_type=jax.ShapeDtypeStruct((NUM_ROWS, EMB), jnp.float32),
    scratch_types=[
        pltpu.VMEM((RPW,), jnp.int32),
        pltpu.VMEM((NUM_SEG, EMB), jnp.float32),
        pltpu.SemaphoreType.DMA,
        pltpu.SemaphoreType.DMA,
    ],
)
def _seg_gather(seg_hbm, table_hbm, out_hbm, idx_v, table_v, sem, ssem):
    wid = lax.axis_index("s") * NC + lax.axis_index("c")
    base = wid * RPW

    pltpu.async_copy(seg_hbm.at[pl.ds(base, RPW)], idx_v, ssem)
    pltpu.async_copy(table_hbm, table_v, ssem)
    pltpu.make_async_copy(seg_hbm.at[pl.ds(base, RPW)], idx_v, ssem).wait()
    pltpu.make_async_copy(table_hbm, table_v, ssem).wait()

    def issue_group(g):
        v = idx_v[pl.ds(g * GRP, GRP)]
        for j in range(GRP):
            pltpu.async_copy(table_v.at[v[j]], out_hbm.at[base + g * GRP + j],
                             sem)

    def wait_group(_g, _):
        pltpu.make_async_copy(out_hbm.at[pl.ds(base, GRP)],
                              out_hbm.at[pl.ds(base, GRP)], sem).wait()
        return 0

    def step(g, _):
        issue_group(g)
        return lax.cond(g >= LAG, lambda: wait_group(g, 0), lambda: 0)

    lax.fori_loop(0, NG, step, 0, unroll=False)
    lax.fori_loop(0, LAG, wait_group, 0, unroll=False)


def kernel(unused, seg, table):
    del unused
    out = _seg_gather(seg.reshape(NUM_ROWS), table)
    return out.reshape(BATCH, SEQ, EMB)
